# baseline (speedup 1.0000x reference)
import jax
import jax.numpy as jnp
import numpy as np
from jax.experimental import pallas as pl

_GRID = 0.5
_SCALE_MULT = 3.0
_PC_MIN = np.array([-50.0, -50.0, -5.0], dtype=np.float32)

_PB = 512


def _agg_kernel(pa_ref, ga_ref, sem_ref, out_ref):
    pa = pa_ref[...]
    ga = ga_ref[...]
    px, py, pz = pa[:, 0:1], pa[:, 1:2], pa[:, 2:3]
    ix, iy, iz = pa[:, 3:4], pa[:, 4:5], pa[:, 5:6]
    mx, my, mz = ga[0:1, :], ga[1:2, :], ga[2:3, :]
    jx, jy, jz = ga[3:4, :], ga[4:5, :], ga[5:6, :]
    rr = ga[6:7, :]
    ax, ay, az = ga[7:8, :], ga[8:9, :], ga[9:10, :]
    c0 = ga[10:11, :]
    dx = px - mx
    dy = py - my
    dz = pz - mz
    power = ax * (dx * dx) + ay * (dy * dy) + az * (dz * dz) + c0
    inside = ((jnp.abs(ix - jx) <= rr)
              & (jnp.abs(iy - jy) <= rr)
              & (jnp.abs(iz - jz) <= rr))
    w = jnp.where(inside, jnp.exp(power), 0.0)
    out_ref[...] = jnp.dot(w, sem_ref[...], preferred_element_type=jnp.float32)


def kernel(pts, means3D, opacities, semantics, scales, cov3D):
    p = pts[0]
    m = means3D[0].astype(jnp.float32)
    op = opacities[0].astype(jnp.float32)
    sem = semantics[0].astype(jnp.float32)
    sc = scales[0]
    cov = cov3D[0].astype(jnp.float32)
    pc_min = jnp.asarray(_PC_MIN)

    pint = ((p - pc_min) / _GRID).astype(jnp.int32).astype(jnp.float32)
    mint = ((m - pc_min) / _GRID).astype(jnp.int32).astype(jnp.float32)
    radii = jnp.ceil(jnp.max(sc, axis=-1) * _SCALE_MULT / _GRID)

    n_pts = p.shape[0]
    n_g = m.shape[0]
    pa = jnp.zeros((n_pts, 8), jnp.float32)
    pa = pa.at[:, 0:3].set(p).at[:, 3:6].set(pint)

    key = (pint[:, 0].astype(jnp.int32) // 16) * 32 + (pint[:, 1].astype(jnp.int32) // 16)
    perm = jnp.argsort(key)
    pa = pa[perm]

    cdiag = jnp.stack([cov[:, 0, 0], cov[:, 1, 1], cov[:, 2, 2]], axis=0)
    ga = jnp.zeros((16, n_g), jnp.float32)
    ga = (ga.at[0:3, :].set(m.T)
            .at[3:6, :].set(mint.T)
            .at[6, :].set(radii)
            .at[7:10, :].set(-0.5 * cdiag)
            .at[10, :].set(jnp.log(op)))

    grid = n_pts // _PB
    out = pl.pallas_call(
        _agg_kernel,
        grid=(grid,),
        in_specs=[
            pl.BlockSpec((_PB, 8), lambda i: (i, 0)),
            pl.BlockSpec((16, n_g), lambda i: (0, 0)),
            pl.BlockSpec((n_g, sem.shape[1]), lambda i: (0, 0)),
        ],
        out_specs=pl.BlockSpec((_PB, sem.shape[1]), lambda i: (i, 0)),
        out_shape=jax.ShapeDtypeStruct((n_pts, sem.shape[1]), jnp.float32),
    )(pa, ga, sem)
    out = jnp.zeros_like(out).at[perm].set(out)
    return out


# speedup vs baseline: 1.9325x; 1.9325x over previous
import jax
import jax.numpy as jnp
import numpy as np
from jax.experimental import pallas as pl

_GRID = 0.5
_SCALE_MULT = 3.0
_PC_MIN = np.array([-50.0, -50.0, -5.0], dtype=np.float32)

_PB = 1024


def _agg_kernel(pa_ref, ga_ref, sem_ref, out_ref):
    pa = pa_ref[...]
    ga = ga_ref[...]
    px, py, pz = pa[:, 0:1], pa[:, 1:2], pa[:, 2:3]
    ix, iy, iz = pa[:, 3:4], pa[:, 4:5], pa[:, 5:6]
    mx, my, mz = ga[0:1, :], ga[1:2, :], ga[2:3, :]
    jx, jy, jz = ga[3:4, :], ga[4:5, :], ga[5:6, :]
    rr = ga[6:7, :]
    ax, ay, az = ga[7:8, :], ga[8:9, :], ga[9:10, :]
    c0 = ga[10:11, :]
    dx = px - mx
    dy = py - my
    dz = pz - mz
    power = ax * (dx * dx) + ay * (dy * dy) + az * (dz * dz) + c0
    inside = ((jnp.abs(ix - jx) <= rr)
              & (jnp.abs(iy - jy) <= rr)
              & (jnp.abs(iz - jz) <= rr))
    w = jnp.where(inside, jnp.exp(power), 0.0)
    out_ref[...] = jnp.dot(w, sem_ref[...], preferred_element_type=jnp.float32)


def kernel(pts, means3D, opacities, semantics, scales, cov3D):
    p = pts[0]
    m = means3D[0].astype(jnp.float32)
    op = opacities[0].astype(jnp.float32)
    sem = semantics[0].astype(jnp.float32)
    sc = scales[0]
    cov = cov3D[0].astype(jnp.float32)
    pc_min = jnp.asarray(_PC_MIN)

    pint = ((p - pc_min) / _GRID).astype(jnp.int32).astype(jnp.float32)
    mint = ((m - pc_min) / _GRID).astype(jnp.int32).astype(jnp.float32)
    radii = jnp.ceil(jnp.max(sc, axis=-1) * _SCALE_MULT / _GRID)

    n_pts = p.shape[0]
    n_g = m.shape[0]
    pa = jnp.zeros((n_pts, 8), jnp.float32)
    pa = pa.at[:, 0:3].set(p).at[:, 3:6].set(pint)

    cdiag = jnp.stack([cov[:, 0, 0], cov[:, 1, 1], cov[:, 2, 2]], axis=0)
    ga = jnp.zeros((16, n_g), jnp.float32)
    ga = (ga.at[0:3, :].set(m.T)
            .at[3:6, :].set(mint.T)
            .at[6, :].set(radii)
            .at[7:10, :].set(-0.5 * cdiag)
            .at[10, :].set(jnp.log(op)))

    grid = n_pts // _PB
    out = pl.pallas_call(
        _agg_kernel,
        grid=(grid,),
        in_specs=[
            pl.BlockSpec((_PB, 8), lambda i: (i, 0)),
            pl.BlockSpec((16, n_g), lambda i: (0, 0)),
            pl.BlockSpec((n_g, sem.shape[1]), lambda i: (0, 0)),
        ],
        out_specs=pl.BlockSpec((_PB, sem.shape[1]), lambda i: (i, 0)),
        out_shape=jax.ShapeDtypeStruct((n_pts, sem.shape[1]), jnp.float32),
    )(pa, ga, sem)
    return out
